# baseline (speedup 1.0000x reference)
import jax
import jax.numpy as jnp
from jax.experimental import pallas as pl

_N = 100
_PAIRS = _N * (_N - 1) // 2


def _body(x_ref, o_ref):
    x = x_ref[...]
    off = 0
    for i in range(1, _N):
        o_ref[:, off:off + i] = x[:, i:i + 1] * x[:, :i]
        off += i


def kernel(x):
    B = x.shape[0]
    tile = 256
    return pl.pallas_call(
        _body,
        grid=(B // tile,),
        in_specs=[pl.BlockSpec((tile, _N), lambda b: (b, 0))],
        out_specs=pl.BlockSpec((tile, _PAIRS), lambda b: (b, 0)),
        out_shape=jax.ShapeDtypeStruct((B, _PAIRS), jnp.float32),
    )(x)
